# Initial kernel scaffold; baseline (speedup 1.0000x reference)
import jax
import jax.numpy as jnp
from jax.experimental import pallas as pl


def kernel(x, edge_index, batch, W0, b0, g0, be0, W1, b1, g1, be1, W2, b2, g2, be2, lin1_W, lin1_b, lin2_W, lin2_b):
    raise NotImplementedError("write your pallas kernel here")


# speedup vs baseline: 18.3501x; 18.3501x over previous
import functools

import jax
import jax.numpy as jnp
from jax import lax
from jax.experimental import pallas as pl
from jax.experimental.pallas import tpu as pltpu
from jax.experimental.pallas import tpu_sc as plsc

N = 10000
E = 320000
F_IN = 128
H = 64
B_GRAPHS = 64
EPS = 1e-5

NC = 2
NS = 16
NW = NC * NS
C = 128
EW = -(-E // NW)
NCHUNK = -(-EW // C)
E_PAD = NW * NCHUNK * C
N_PAD = 10112
HP = 64
ROWS_PER_TILE = N_PAD // NS
HIGH = lax.Precision.HIGHEST

_MESH = dict(core_axis_name="c", subcore_axis_name="s", num_cores=NC,
             num_subcores=NS)


def _deg_body(dst_hbm, zeros_hbm, ones_hbm, out_hbm, idx_v, ones_v, acc, sem):
    del sem
    c = lax.axis_index("c")
    s = lax.axis_index("s")
    wid = c * NS + s
    pltpu.sync_copy(dst_hbm.at[wid], idx_v)
    pltpu.sync_copy(ones_hbm, ones_v)
    pltpu.sync_copy(zeros_hbm.at[pl.ds(s * ROWS_PER_TILE, ROWS_PER_TILE)],
                    acc.at[pl.ds(s * ROWS_PER_TILE, ROWS_PER_TILE)])
    plsc.subcore_barrier()

    @pl.loop(0, NCHUNK)
    def _(j):
        pltpu.sync_copy(ones_v, acc.at[idx_v.at[j]], add=True)

    plsc.subcore_barrier()
    pltpu.sync_copy(acc.at[pl.ds(s * ROWS_PER_TILE, ROWS_PER_TILE)],
                    out_hbm.at[c, pl.ds(s * ROWS_PER_TILE, ROWS_PER_TILE)])


def _sc_degree(dst_p, zeros8, ones8):
    return pl.kernel(
        _deg_body,
        out_type=jax.ShapeDtypeStruct((NC, N_PAD, 8), jnp.float32),
        mesh=plsc.VectorSubcoreMesh(**_MESH),
        compiler_params=pltpu.CompilerParams(use_tc_tiling_on_sc=False),
        scratch_types=[
            pltpu.VMEM((NCHUNK, C), jnp.int32),
            pltpu.VMEM((C, 8), jnp.float32),
            pltpu.VMEM_SHARED((N_PAD, 8), jnp.float32),
            pltpu.SemaphoreType.DMA,
        ],
    )(dst_p, zeros8, ones8)


def _agg_body(src_hbm, dst_hbm, hp_hbm, zeros_hbm, out_hbm,
              sidx_v, didx_v, gbuf0, gbuf1, acc, sem0, sem1):
    c = lax.axis_index("c")
    s = lax.axis_index("s")
    wid = c * NS + s
    pltpu.sync_copy(src_hbm.at[wid], sidx_v)
    pltpu.sync_copy(dst_hbm.at[wid], didx_v)
    pltpu.sync_copy(zeros_hbm.at[pl.ds(s * ROWS_PER_TILE, ROWS_PER_TILE)],
                    acc.at[pl.ds(s * ROWS_PER_TILE, ROWS_PER_TILE)])
    plsc.subcore_barrier()

    pltpu.async_copy(hp_hbm.at[sidx_v.at[0]], gbuf0, sem0)

    @pl.loop(0, NCHUNK)
    def _(j):
        even = j % 2 == 0

        @pl.when(even)
        def _():
            pltpu.make_async_copy(hp_hbm.at[sidx_v.at[j]], gbuf0, sem0).wait()

            @pl.when(j + 1 < NCHUNK)
            def _():
                pltpu.async_copy(hp_hbm.at[sidx_v.at[j + 1]], gbuf1, sem1)

            pltpu.sync_copy(gbuf0, acc.at[didx_v.at[j]], add=True)

        @pl.when(jnp.logical_not(even))
        def _():
            pltpu.make_async_copy(hp_hbm.at[sidx_v.at[j]], gbuf1, sem1).wait()

            @pl.when(j + 1 < NCHUNK)
            def _():
                pltpu.async_copy(hp_hbm.at[sidx_v.at[j + 1]], gbuf0, sem0)

            pltpu.sync_copy(gbuf1, acc.at[didx_v.at[j]], add=True)

    plsc.subcore_barrier()
    pltpu.sync_copy(acc.at[pl.ds(s * ROWS_PER_TILE, ROWS_PER_TILE)],
                    out_hbm.at[c, pl.ds(s * ROWS_PER_TILE, ROWS_PER_TILE)])


def _sc_aggregate(src_p, dst_p, hp, zerosH):
    return pl.kernel(
        _agg_body,
        out_type=jax.ShapeDtypeStruct((NC, N_PAD, HP), jnp.float32),
        mesh=plsc.VectorSubcoreMesh(**_MESH),
        compiler_params=pltpu.CompilerParams(use_tc_tiling_on_sc=False),
        scratch_types=[
            pltpu.VMEM((NCHUNK, C), jnp.int32),
            pltpu.VMEM((NCHUNK, C), jnp.int32),
            pltpu.VMEM((C, HP), jnp.float32),
            pltpu.VMEM((C, HP), jnp.float32),
            pltpu.VMEM_SHARED((N_PAD, HP), jnp.float32),
            pltpu.SemaphoreType.DMA,
            pltpu.SemaphoreType.DMA,
        ],
    )(src_p, dst_p, hp, zerosH)


def _pre_body(x_ref, w_ref, degp_ref, dinv_ref, hp_ref):
    deg = 1.0 + degp_ref[0, :N, 0:1] + degp_ref[1, :N, 0:1]
    dinv = lax.rsqrt(deg)
    dinv_ref[...] = dinv
    h2 = jnp.dot(x_ref[...], w_ref[...], precision=HIGH,
                 preferred_element_type=jnp.float32)
    hp_ref[:, :H] = dinv * h2
    if HP > H:
        hp_ref[:, H:] = jnp.zeros((N, HP - H), jnp.float32)


def _tc_pre(x, w0, degp):
    return pl.pallas_call(
        _pre_body,
        out_shape=(jax.ShapeDtypeStruct((N, 1), jnp.float32),
                   jax.ShapeDtypeStruct((N, HP), jnp.float32)),
    )(x, w0, degp)


def _bn_relu(agg, g, be):
    mean = jnp.mean(agg, axis=0, keepdims=True)
    ctr = agg - mean
    var = jnp.mean(ctr * ctr, axis=0, keepdims=True)
    return jnp.maximum(ctr * lax.rsqrt(var + EPS) * g + be, 0.0)


def _mid_body(parts_ref, hp_ref, dinv_ref, b_ref, g_ref, be_ref, w_ref,
              out_ref):
    dinv = dinv_ref[...]
    agg = dinv * (parts_ref[0, :N, :H] + parts_ref[1, :N, :H]
                  + hp_ref[:, :H]) + b_ref[...]
    hr = _bn_relu(agg, g_ref[...], be_ref[...])
    h2 = jnp.dot(hr, w_ref[...], precision=HIGH,
                 preferred_element_type=jnp.float32)
    out_ref[:, :H] = dinv * h2
    if HP > H:
        out_ref[:, H:] = jnp.zeros((N, HP - H), jnp.float32)


def _tc_mid(parts, hp, dinv, b, g, be, w_next):
    return pl.pallas_call(
        _mid_body,
        out_shape=jax.ShapeDtypeStruct((N, HP), jnp.float32),
    )(parts, hp, dinv, b.reshape(1, H), g.reshape(1, H), be.reshape(1, H),
      w_next)


def _fin_body(parts_ref, hp_ref, dinv_ref, b_ref, g_ref, be_ref, batch_ref,
              l1w_ref, l1b_ref, l2w_ref, l2b_ref, out_ref):
    agg = dinv_ref[...] * (parts_ref[0, :N, :H] + parts_ref[1, :N, :H]
                           + hp_ref[:, :H]) + b_ref[...]
    h3 = _bn_relu(agg, g_ref[...], be_ref[...])
    gid = lax.broadcasted_iota(jnp.int32, (N, B_GRAPHS), 1)
    onehot = (batch_ref[...] == gid).astype(jnp.float32)
    sums = lax.dot_general(onehot, h3, (((0,), (0,)), ((), ())),
                           precision=HIGH,
                           preferred_element_type=jnp.float32)
    counts = jnp.sum(onehot, axis=0)[:, None]
    pooled = sums / jnp.maximum(counts, 1.0)
    hh = jnp.maximum(jnp.dot(pooled, l1w_ref[...], precision=HIGH,
                             preferred_element_type=jnp.float32)
                     + l1b_ref[...], 0.0)
    out_ref[...] = jnp.dot(hh, l2w_ref[...], precision=HIGH,
                           preferred_element_type=jnp.float32) + l2b_ref[...]


def _tc_final(parts, hp, dinv, b, g, be, batch2d, l1w, l1b, l2w, l2b):
    return pl.pallas_call(
        _fin_body,
        out_shape=jax.ShapeDtypeStruct((B_GRAPHS, 1), jnp.float32),
    )(parts, hp, dinv, b.reshape(1, H), g.reshape(1, H), be.reshape(1, H),
      batch2d, l1w, l1b.reshape(1, -1), l2w, l2b.reshape(1, -1))


def kernel(x, edge_index, batch, W0, b0, g0, be0, W1, b1, g1, be1,
           W2, b2, g2, be2, lin1_W, lin1_b, lin2_W, lin2_b):
    src = edge_index[0].astype(jnp.int32)
    dst = edge_index[1].astype(jnp.int32)
    pad = E_PAD - E
    src_p = jnp.concatenate([src, jnp.zeros((pad,), jnp.int32)])
    dst_p = jnp.concatenate([dst, jnp.full((pad,), N, jnp.int32)])
    src_p = src_p.reshape(NW, NCHUNK, C)
    dst_p = dst_p.reshape(NW, NCHUNK, C)

    zerosH = jnp.zeros((N_PAD, HP), jnp.float32)
    zeros8 = jnp.zeros((N_PAD, 8), jnp.float32)
    ones8 = jnp.ones((C, 8), jnp.float32)

    degp = _sc_degree(dst_p, zeros8, ones8)
    dinv, hp = _tc_pre(x, W0, degp)

    parts = _sc_aggregate(src_p, dst_p, hp, zerosH)
    hp = _tc_mid(parts, hp, dinv, b0, g0, be0, W1)

    parts = _sc_aggregate(src_p, dst_p, hp, zerosH)
    hp = _tc_mid(parts, hp, dinv, b1, g1, be1, W2)

    parts = _sc_aggregate(src_p, dst_p, hp, zerosH)
    batch2d = batch.astype(jnp.int32).reshape(N, 1)
    return _tc_final(parts, hp, dinv, b2, g2, be2, batch2d,
                     lin1_W, lin1_b, lin2_W, lin2_b)
